# baseline (speedup 1.0000x reference)
import functools

import jax
import jax.numpy as jnp
from jax import lax
from jax.experimental import pallas as pl
from jax.experimental.pallas import tpu as pltpu
from jax.experimental.pallas import tpu_sc as plsc

VOCAB = 1000000
EMB = 64
HID = 128
BATCH = 16384
SEQ = 200

_NC, _NS = 2, 16
_NW = _NC * _NS
_BPW = BATCH // _NW
_G = 64
_NG = _BPW // _G
_S0 = 128
_S1 = SEQ - _S0


def _pool_body(x_hbm, emb_hbm, out_hbm, idx_v, rows_a, rows_b, pooled_v,
               sem_a, sem_b):
    wid = lax.axis_index("s") * _NC + lax.axis_index("c")
    base = wid * _BPW

    def copies(e, rows_ref, sem):
        return (
            pltpu.make_async_copy(
                emb_hbm.at[idx_v.at[e, pl.ds(0, _S0)]],
                rows_ref.at[pl.ds(0, _S0), :], sem),
            pltpu.make_async_copy(
                emb_hbm.at[idx_v.at[e, pl.ds(_S0, _S1)]],
                rows_ref.at[pl.ds(_S0, _S1), :], sem),
        )

    def start(e, rows_ref, sem):
        for cp in copies(e, rows_ref, sem):
            cp.start()

    def wait(e, rows_ref, sem):
        for cp in copies(e, rows_ref, sem):
            cp.wait()

    def accum_into(rows_ref, e):
        def body(j, acc):
            return tuple(acc[k] + rows_ref[j, pl.ds(32 * k, 32)]
                         for k in range(EMB // 32))

        acc = lax.fori_loop(
            0, SEQ, body,
            tuple(jnp.zeros((32,), jnp.bfloat16) for _ in range(EMB // 32)),
            unroll=4)
        inv = jnp.bfloat16(1.0 / SEQ)
        for k in range(EMB // 32):
            pooled_v[e, pl.ds(32 * k, 32)] = acc[k] * inv

    def group(gi, carry):
        g0 = base + gi * _G
        pltpu.sync_copy(x_hbm.at[pl.ds(g0, _G), :], idx_v)
        start(0, rows_a, sem_a)

        def pair(p, c):
            e = 2 * p
            start(e + 1, rows_b, sem_b)
            wait(e, rows_a, sem_a)
            accum_into(rows_a, e)

            @pl.when(p + 1 < _G // 2)
            def _():
                start(e + 2, rows_a, sem_a)

            wait(e + 1, rows_b, sem_b)
            accum_into(rows_b, e + 1)
            return c

        lax.fori_loop(0, _G // 2, pair, 0)
        pltpu.sync_copy(pooled_v, out_hbm.at[pl.ds(g0, _G), :])
        return carry

    lax.fori_loop(0, _NG, group, 0)


_pool = functools.partial(
    pl.kernel,
    mesh=plsc.VectorSubcoreMesh(core_axis_name="c", subcore_axis_name="s"),
    out_type=jax.ShapeDtypeStruct((BATCH, EMB), jnp.bfloat16),
    scratch_types=[
        pltpu.VMEM((_G, SEQ), jnp.int32),
        pltpu.VMEM((SEQ, EMB), jnp.bfloat16),
        pltpu.VMEM((SEQ, EMB), jnp.bfloat16),
        pltpu.VMEM((_G, EMB), jnp.bfloat16),
        pltpu.SemaphoreType.DMA,
        pltpu.SemaphoreType.DMA,
    ],
    compiler_params=pltpu.CompilerParams(use_tc_tiling_on_sc=False),
)(_pool_body)


_BBLK = 2048


def _mlp_body(p_ref, w1_ref, b1_ref, w2_ref, b2_ref, o_ref):
    p = p_ref[...].astype(jnp.float32)
    h = jnp.dot(p, w1_ref[...], preferred_element_type=jnp.float32)
    h = jnp.maximum(h + b1_ref[...], 0.0)
    z = jnp.dot(h, w2_ref[...], preferred_element_type=jnp.float32) + b2_ref[...]
    o_ref[...] = 1.0 / (1.0 + jnp.exp(-z))


def _mlp(pooled, W1, b1, W2, b2):
    return pl.pallas_call(
        _mlp_body,
        grid=(BATCH // _BBLK,),
        in_specs=[
            pl.BlockSpec((_BBLK, EMB), lambda i: (i, 0)),
            pl.BlockSpec((EMB, HID), lambda i: (0, 0)),
            pl.BlockSpec((1, HID), lambda i: (0, 0)),
            pl.BlockSpec((HID, 1), lambda i: (0, 0)),
            pl.BlockSpec((1, 1), lambda i: (0, 0)),
        ],
        out_specs=pl.BlockSpec((_BBLK, 1), lambda i: (i, 0)),
        out_shape=jax.ShapeDtypeStruct((BATCH, 1), jnp.float32),
    )(pooled, W1, b1.reshape(1, HID), W2, b2.reshape(1, 1))


def kernel(x, emb, W1, b1, W2, b2):
    pooled = _pool(x.astype(jnp.int32), emb.astype(jnp.bfloat16))
    return _mlp(pooled, W1, b1, W2, b2)


# speedup vs baseline: 1.1783x; 1.1783x over previous
import functools

import jax
import jax.numpy as jnp
import numpy as np
from jax import lax
from jax.experimental import pallas as pl
from jax.experimental.pallas import tpu as pltpu
from jax.experimental.pallas import tpu_sc as plsc

VOCAB = 1000000
EMB = 64
HID = 128
BATCH = 16384
SEQ = 200

_NC, _NS = 2, 16
_NW = _NC * _NS
_BPW = BATCH // _NW
_G = 64
_NG = _BPW // _G
_S0 = 128
_S1 = SEQ - _S0

_PERM = np.zeros(EMB, dtype=np.int32)
for _c in range(EMB):
    _half, _cc = divmod(_c, 32)
    _w = 16 * _half + _cc // 2
    _PERM[_c] = _w + 32 * (_cc % 2)

_PC = 6400


def _prep_body(et_ref, o_ref):
    t = jnp.transpose(et_ref[...]).astype(jnp.bfloat16)
    lo = lax.bitcast_convert_type(t[:, :32], jnp.uint16).astype(jnp.uint32)
    hi = lax.bitcast_convert_type(t[:, 32:], jnp.uint16).astype(jnp.uint32)
    o_ref[...] = lax.bitcast_convert_type(lo | (hi << 16), jnp.float32)


def _prep(et):
    return pl.pallas_call(
        _prep_body,
        grid=(pl.cdiv(VOCAB, _PC),),
        in_specs=[pl.BlockSpec((EMB, _PC), lambda i: (0, i))],
        out_specs=pl.BlockSpec((_PC, 32), lambda i: (i, 0)),
        out_shape=jax.ShapeDtypeStruct((VOCAB, 32), jnp.float32),
    )(et)


def _pool_body(x_hbm, emb_hbm, out_hbm, idx_v, rows_a, rows_b, pooled_v,
               sem_a, sem_b):
    wid = lax.axis_index("s") * _NC + lax.axis_index("c")
    base = wid * _BPW

    def copies(e, rows_ref, sem):
        return (
            pltpu.make_async_copy(
                emb_hbm.at[idx_v.at[e, pl.ds(0, _S0)]],
                rows_ref.at[pl.ds(0, _S0), :], sem),
            pltpu.make_async_copy(
                emb_hbm.at[idx_v.at[e, pl.ds(_S0, _S1)]],
                rows_ref.at[pl.ds(_S0, _S1), :], sem),
        )

    def start(e, rows_ref, sem):
        for cp in copies(e, rows_ref, sem):
            cp.start()

    def wait(e, rows_ref, sem):
        for cp in copies(e, rows_ref, sem):
            cp.wait()

    def accum_into(rows_ref, e):
        def body(j, acc):
            return tuple(
                acc[k] + plsc.bitcast(rows_ref[j, pl.ds(16 * k, 16)],
                                      jnp.bfloat16)
                for k in range(2))

        acc = lax.fori_loop(
            0, SEQ, body,
            tuple(jnp.zeros((32,), jnp.bfloat16) for _ in range(2)),
            unroll=4)
        inv = jnp.bfloat16(1.0 / SEQ)
        for k in range(2):
            pooled_v[e, pl.ds(32 * k, 32)] = acc[k] * inv

    def group(gi, carry):
        g0 = base + gi * _G
        pltpu.sync_copy(x_hbm.at[pl.ds(g0, _G), :], idx_v)
        start(0, rows_a, sem_a)

        def pair(p, c):
            e = 2 * p
            start(e + 1, rows_b, sem_b)
            wait(e, rows_a, sem_a)
            accum_into(rows_a, e)

            @pl.when(p + 1 < _G // 2)
            def _():
                start(e + 2, rows_a, sem_a)

            wait(e + 1, rows_b, sem_b)
            accum_into(rows_b, e + 1)
            return c

        lax.fori_loop(0, _G // 2, pair, 0)
        pltpu.sync_copy(pooled_v, out_hbm.at[pl.ds(g0, _G), :])
        return carry

    lax.fori_loop(0, _NG, group, 0)


_pool = functools.partial(
    pl.kernel,
    mesh=plsc.VectorSubcoreMesh(core_axis_name="c", subcore_axis_name="s"),
    out_type=jax.ShapeDtypeStruct((BATCH, EMB), jnp.bfloat16),
    scratch_types=[
        pltpu.VMEM((_G, SEQ), jnp.int32),
        pltpu.VMEM((SEQ, 32), jnp.float32),
        pltpu.VMEM((SEQ, 32), jnp.float32),
        pltpu.VMEM((_G, EMB), jnp.bfloat16),
        pltpu.SemaphoreType.DMA,
        pltpu.SemaphoreType.DMA,
    ],
    compiler_params=pltpu.CompilerParams(use_tc_tiling_on_sc=False,
                                         needs_layout_passes=False),
)(_pool_body)


_BBLK = 2048


def _mlp_body(p_ref, w1_ref, b1_ref, w2_ref, b2_ref, o_ref):
    p = p_ref[...].astype(jnp.float32)
    h = jnp.dot(p, w1_ref[...], preferred_element_type=jnp.float32)
    h = jnp.maximum(h + b1_ref[...], 0.0)
    z = jnp.dot(h, w2_ref[...], preferred_element_type=jnp.float32) + b2_ref[...]
    o_ref[...] = 1.0 / (1.0 + jnp.exp(-z))


def _mlp(pooled, W1p, b1, W2, b2):
    return pl.pallas_call(
        _mlp_body,
        grid=(BATCH // _BBLK,),
        in_specs=[
            pl.BlockSpec((_BBLK, EMB), lambda i: (i, 0)),
            pl.BlockSpec((EMB, HID), lambda i: (0, 0)),
            pl.BlockSpec((1, HID), lambda i: (0, 0)),
            pl.BlockSpec((HID, 1), lambda i: (0, 0)),
            pl.BlockSpec((1, 1), lambda i: (0, 0)),
        ],
        out_specs=pl.BlockSpec((_BBLK, 1), lambda i: (i, 0)),
        out_shape=jax.ShapeDtypeStruct((BATCH, 1), jnp.float32),
    )(pooled, W1p, b1.reshape(1, HID), W2, b2.reshape(1, 1))


def kernel(x, emb, W1, b1, W2, b2):
    embp = _prep(emb.T)
    pooled = _pool(x.astype(jnp.int32), embp)
    return _mlp(pooled, W1[jnp.asarray(_PERM)], b1, W2, b2)


# speedup vs baseline: 1.3813x; 1.1723x over previous
import functools

import jax
import jax.numpy as jnp
import numpy as np
from jax import lax
from jax.experimental import pallas as pl
from jax.experimental.pallas import tpu as pltpu
from jax.experimental.pallas import tpu_sc as plsc

VOCAB = 1000000
EMB = 64
HID = 128
BATCH = 16384
SEQ = 200

_NC, _NS = 2, 16
_NW = _NC * _NS
_BPW = BATCH // _NW
_G = 64
_NG = _BPW // _G
_S0 = 128
_S1 = SEQ - _S0

_PERM = np.zeros(EMB, dtype=np.int32)
for _c in range(EMB):
    _half, _cc = divmod(_c, 32)
    _w = 16 * _half + _cc // 2
    _PERM[_c] = _w + 32 * (_cc % 2)

_PC = 6400


def _prep_body(et_ref, o_ref):
    t = jnp.transpose(et_ref[...]).astype(jnp.bfloat16)
    lo = lax.bitcast_convert_type(t[:, :32], jnp.uint16).astype(jnp.uint32)
    hi = lax.bitcast_convert_type(t[:, 32:], jnp.uint16).astype(jnp.uint32)
    w = lax.bitcast_convert_type(lo | (hi << 16), jnp.float32)
    w4 = w.reshape(_PC // 4, 4, 32)
    o_ref[...] = jnp.concatenate([w4[:, q, :] for q in range(4)], axis=1)


def _prep(et):
    return pl.pallas_call(
        _prep_body,
        grid=(pl.cdiv(VOCAB, _PC),),
        in_specs=[pl.BlockSpec((EMB, _PC), lambda i: (0, i))],
        out_specs=pl.BlockSpec((_PC // 4, 128), lambda i: (i, 0)),
        out_shape=jax.ShapeDtypeStruct((VOCAB // 4, 128), jnp.float32),
    )(et)


def _pool_body(x_hbm, emb_hbm, out_hbm, idx_v, rows_a, rows_b, pooled_v,
               sem_a, sem_b):
    wid = lax.axis_index("s") * _NC + lax.axis_index("c")
    base = wid * _BPW

    def copies(e, rows_ref, sem):
        return (
            pltpu.make_async_copy(
                emb_hbm.at[idx_v.at[e, pl.ds(0, _S0)]],
                rows_ref.at[pl.ds(0, _S0), :], sem),
            pltpu.make_async_copy(
                emb_hbm.at[idx_v.at[e, pl.ds(_S0, _S1)]],
                rows_ref.at[pl.ds(_S0, _S1), :], sem),
        )

    def start(e, rows_ref, sem):
        for cp in copies(e, rows_ref, sem):
            cp.start()

    def wait(e, rows_ref, sem):
        for cp in copies(e, rows_ref, sem):
            cp.wait()

    def accum_into(rows_ref, e):
        def body(j, acc):
            return tuple(
                acc[k] + plsc.bitcast(rows_ref[j, pl.ds(16 * k, 16)],
                                      jnp.bfloat16)
                for k in range(2))

        acc = lax.fori_loop(
            0, SEQ, body,
            tuple(jnp.zeros((32,), jnp.bfloat16) for _ in range(2)),
            unroll=4)
        inv = jnp.bfloat16(1.0 / SEQ)
        for k in range(2):
            pooled_v[e, pl.ds(32 * k, 32)] = acc[k] * inv

    def group(gi, carry):
        g0 = base + gi * _G
        pltpu.sync_copy(x_hbm.at[pl.ds(g0, _G), :], idx_v)
        start(0, rows_a, sem_a)

        def pair(p, c):
            e = 2 * p
            start(e + 1, rows_b, sem_b)
            wait(e, rows_a, sem_a)
            accum_into(rows_a, e)

            @pl.when(p + 1 < _G // 2)
            def _():
                start(e + 2, rows_a, sem_a)

            wait(e + 1, rows_b, sem_b)
            accum_into(rows_b, e + 1)
            return c

        lax.fori_loop(0, _G // 2, pair, 0)
        pltpu.sync_copy(pooled_v, out_hbm.at[pl.ds(g0, _G), :])
        return carry

    lax.fori_loop(0, _NG, group, 0)


_pool = functools.partial(
    pl.kernel,
    mesh=plsc.VectorSubcoreMesh(core_axis_name="c", subcore_axis_name="s"),
    out_type=jax.ShapeDtypeStruct((BATCH, EMB), jnp.bfloat16),
    scratch_types=[
        pltpu.VMEM((_G, SEQ), jnp.int32),
        pltpu.VMEM((SEQ, 32), jnp.float32),
        pltpu.VMEM((SEQ, 32), jnp.float32),
        pltpu.VMEM((_G, EMB), jnp.bfloat16),
        pltpu.SemaphoreType.DMA,
        pltpu.SemaphoreType.DMA,
    ],
    compiler_params=pltpu.CompilerParams(use_tc_tiling_on_sc=False,
                                         needs_layout_passes=False),
)(_pool_body)


_BBLK = 2048


def _mlp_body(p_ref, w1_ref, b1_ref, w2_ref, b2_ref, o_ref):
    p = p_ref[...].astype(jnp.float32)
    h = jnp.dot(p, w1_ref[...], preferred_element_type=jnp.float32)
    h = jnp.maximum(h + b1_ref[...], 0.0)
    z = jnp.dot(h, w2_ref[...], preferred_element_type=jnp.float32) + b2_ref[...]
    o_ref[...] = 1.0 / (1.0 + jnp.exp(-z))


def _mlp(pooled, W1p, b1, W2, b2):
    return pl.pallas_call(
        _mlp_body,
        grid=(BATCH // _BBLK,),
        in_specs=[
            pl.BlockSpec((_BBLK, EMB), lambda i: (i, 0)),
            pl.BlockSpec((EMB, HID), lambda i: (0, 0)),
            pl.BlockSpec((1, HID), lambda i: (0, 0)),
            pl.BlockSpec((HID, 1), lambda i: (0, 0)),
            pl.BlockSpec((1, 1), lambda i: (0, 0)),
        ],
        out_specs=pl.BlockSpec((_BBLK, 1), lambda i: (i, 0)),
        out_shape=jax.ShapeDtypeStruct((BATCH, 1), jnp.float32),
    )(pooled, W1p, b1.reshape(1, HID), W2, b2.reshape(1, 1))


def kernel(x, emb, W1, b1, W2, b2):
    embp = _prep(emb.T).reshape(VOCAB, 32)
    pooled = _pool(x.astype(jnp.int32), embp)
    return _mlp(pooled, W1[jnp.asarray(_PERM)], b1, W2, b2)


# speedup vs baseline: 1.7977x; 1.3014x over previous
import functools

import jax
import jax.numpy as jnp
import numpy as np
from jax import lax
from jax.experimental import pallas as pl
from jax.experimental.pallas import tpu as pltpu
from jax.experimental.pallas import tpu_sc as plsc

VOCAB = 1000000
EMB = 64
HID = 128
BATCH = 16384
SEQ = 200

_NC, _NS = 2, 16
_NW = _NC * _NS
_BPW = BATCH // _NW
_G = 64
_NG = _BPW // _G
_S0 = 128
_S1 = SEQ - _S0

_PERM = np.zeros(EMB, dtype=np.int32)
for _c in range(EMB):
    _half, _cc = divmod(_c, 32)
    _w = 16 * _half + _cc // 2
    _PERM[_c] = _w + 32 * (_cc % 2)

_Q = 250880
_VP = 4 * _Q
_PB = 2560
_PGRID = _Q // _PB


def _prep_body(e0_ref, e1_ref, e2_ref, e3_ref, o_ref):
    for q, ref in enumerate((e0_ref, e1_ref, e2_ref, e3_ref)):
        t = jnp.transpose(ref[...]).astype(jnp.bfloat16)
        lo = lax.bitcast_convert_type(t[:, :32], jnp.uint16).astype(jnp.uint32)
        hi = lax.bitcast_convert_type(t[:, 32:], jnp.uint16).astype(jnp.uint32)
        o_ref[:, pl.ds(32 * q, 32)] = lax.bitcast_convert_type(
            lo | (hi << 16), jnp.float32)


def _prep(et):
    last = VOCAB // _PB
    specs = [
        pl.BlockSpec((EMB, _PB), functools.partial(
            lambda q, i: (0, jnp.minimum(i + _PGRID * q, last)), q))
        for q in range(4)
    ]
    return pl.pallas_call(
        _prep_body,
        grid=(_PGRID,),
        in_specs=specs,
        out_specs=pl.BlockSpec((_PB, 128), lambda i: (i, 0)),
        out_shape=jax.ShapeDtypeStruct((_Q, 128), jnp.float32),
    )(et, et, et, et)


def _pool_body(x_hbm, emb_hbm, out_hbm, idx_v, rows_a, rows_b, pooled_v,
               sem_a, sem_b):
    wid = lax.axis_index("s") * _NC + lax.axis_index("c")
    base = wid * _BPW

    def copies(e, rows_ref, sem):
        return (
            pltpu.make_async_copy(
                emb_hbm.at[idx_v.at[e, pl.ds(0, _S0)]],
                rows_ref.at[pl.ds(0, _S0), :], sem),
            pltpu.make_async_copy(
                emb_hbm.at[idx_v.at[e, pl.ds(_S0, _S1)]],
                rows_ref.at[pl.ds(_S0, _S1), :], sem),
        )

    def start(e, rows_ref, sem):
        for cp in copies(e, rows_ref, sem):
            cp.start()

    def wait(e, rows_ref, sem):
        for cp in copies(e, rows_ref, sem):
            cp.wait()

    def accum_into(rows_ref, e):
        def body(j, acc):
            return tuple(
                acc[k] + plsc.bitcast(rows_ref[j, pl.ds(16 * k, 16)],
                                      jnp.bfloat16)
                for k in range(2))

        acc = lax.fori_loop(
            0, SEQ, body,
            tuple(jnp.zeros((32,), jnp.bfloat16) for _ in range(2)),
            unroll=4)
        inv = jnp.bfloat16(1.0 / SEQ)
        for k in range(2):
            pooled_v[e, pl.ds(32 * k, 32)] = acc[k] * inv

    def group(gi, carry):
        g0 = base + gi * _G
        pltpu.sync_copy(x_hbm.at[pl.ds(g0, _G), :], idx_v)
        start(0, rows_a, sem_a)

        def pair(p, c):
            e = 2 * p
            start(e + 1, rows_b, sem_b)
            wait(e, rows_a, sem_a)
            accum_into(rows_a, e)

            @pl.when(p + 1 < _G // 2)
            def _():
                start(e + 2, rows_a, sem_a)

            wait(e + 1, rows_b, sem_b)
            accum_into(rows_b, e + 1)
            return c

        lax.fori_loop(0, _G // 2, pair, 0)
        pltpu.sync_copy(pooled_v, out_hbm.at[pl.ds(g0, _G), :])
        return carry

    lax.fori_loop(0, _NG, group, 0)


_pool = functools.partial(
    pl.kernel,
    mesh=plsc.VectorSubcoreMesh(core_axis_name="c", subcore_axis_name="s"),
    out_type=jax.ShapeDtypeStruct((BATCH, EMB), jnp.bfloat16),
    scratch_types=[
        pltpu.VMEM((_G, SEQ), jnp.int32),
        pltpu.VMEM((SEQ, 32), jnp.float32),
        pltpu.VMEM((SEQ, 32), jnp.float32),
        pltpu.VMEM((_G, EMB), jnp.bfloat16),
        pltpu.SemaphoreType.DMA,
        pltpu.SemaphoreType.DMA,
    ],
    compiler_params=pltpu.CompilerParams(use_tc_tiling_on_sc=False,
                                         needs_layout_passes=False),
)(_pool_body)


_BBLK = 2048


def _mlp_body(p_ref, w1_ref, b1_ref, w2_ref, b2_ref, o_ref):
    p = p_ref[...].astype(jnp.float32)
    h = jnp.dot(p, w1_ref[...], preferred_element_type=jnp.float32)
    h = jnp.maximum(h + b1_ref[...], 0.0)
    z = jnp.dot(h, w2_ref[...], preferred_element_type=jnp.float32) + b2_ref[...]
    o_ref[...] = 1.0 / (1.0 + jnp.exp(-z))


def _mlp(pooled, W1p, b1, W2, b2):
    return pl.pallas_call(
        _mlp_body,
        grid=(BATCH // _BBLK,),
        in_specs=[
            pl.BlockSpec((_BBLK, EMB), lambda i: (i, 0)),
            pl.BlockSpec((EMB, HID), lambda i: (0, 0)),
            pl.BlockSpec((1, HID), lambda i: (0, 0)),
            pl.BlockSpec((HID, 1), lambda i: (0, 0)),
            pl.BlockSpec((1, 1), lambda i: (0, 0)),
        ],
        out_specs=pl.BlockSpec((_BBLK, 1), lambda i: (i, 0)),
        out_shape=jax.ShapeDtypeStruct((BATCH, 1), jnp.float32),
    )(pooled, W1p, b1.reshape(1, HID), W2, b2.reshape(1, 1))


def kernel(x, emb, W1, b1, W2, b2):
    embp = _prep(emb.T).reshape(_VP, 32)
    x32 = x.astype(jnp.int32)
    q = x32 // _Q
    xt = 4 * (x32 - q * _Q) + q
    pooled = _pool(xt, embp)
    return _mlp(pooled, W1[jnp.asarray(_PERM)], b1, W2, b2)


# speedup vs baseline: 2.2230x; 1.2366x over previous
import functools

import jax
import jax.numpy as jnp
import numpy as np
from jax import lax
from jax.experimental import pallas as pl
from jax.experimental.pallas import tpu as pltpu
from jax.experimental.pallas import tpu_sc as plsc

VOCAB = 1000000
EMB = 64
HID = 128
BATCH = 16384
SEQ = 200

_NC, _NS = 2, 16
_NW = _NC * _NS
_BPW = BATCH // _NW
_G = 64
_NG = _BPW // _G
_S0 = 128
_S1 = SEQ - _S0

_PERM = np.zeros(EMB, dtype=np.int32)
for _c in range(EMB):
    _half, _cc = divmod(_c, 32)
    _w = 16 * _half + _cc // 2
    _PERM[_c] = _w + 32 * (_cc % 2)

_Q = 250880
_VP = 4 * _Q
_PB = 2560
_PGRID = _Q // _PB


def _prep_body(e0_ref, e1_ref, e2_ref, e3_ref, o_ref):
    for q, ref in enumerate((e0_ref, e1_ref, e2_ref, e3_ref)):
        t = jnp.transpose(ref[...]).astype(jnp.bfloat16)
        lo = lax.bitcast_convert_type(t[:, :32], jnp.uint16).astype(jnp.uint32)
        hi = lax.bitcast_convert_type(t[:, 32:], jnp.uint16).astype(jnp.uint32)
        o_ref[:, pl.ds(32 * q, 32)] = lax.bitcast_convert_type(
            lo | (hi << 16), jnp.float32)


def _prep(et):
    last = VOCAB // _PB
    specs = [
        pl.BlockSpec((EMB, _PB), functools.partial(
            lambda q, i: (0, jnp.minimum(i + _PGRID * q, last)), q))
        for q in range(4)
    ]
    return pl.pallas_call(
        _prep_body,
        grid=(_PGRID,),
        in_specs=specs,
        out_specs=pl.BlockSpec((_PB, 128), lambda i: (i, 0)),
        out_shape=jax.ShapeDtypeStruct((_Q, 128), jnp.float32),
    )(et, et, et, et)


def _pool_body(x_hbm, emb_hbm, out_hbm, idx_v, rows_a, rows_b, rows_c,
               rows_d, pooled_v, sem_a, sem_b, sem_c, sem_d):
    wid = lax.axis_index("s") * _NC + lax.axis_index("c")
    base = wid * _BPW

    def copies(e, rows_ref, sem):
        return (
            pltpu.make_async_copy(
                emb_hbm.at[idx_v.at[e, pl.ds(0, _S0)]],
                rows_ref.at[pl.ds(0, _S0), :], sem),
            pltpu.make_async_copy(
                emb_hbm.at[idx_v.at[e, pl.ds(_S0, _S1)]],
                rows_ref.at[pl.ds(_S0, _S1), :], sem),
        )

    def start(e, rows_ref, sem):
        for cp in copies(e, rows_ref, sem):
            cp.start()

    def wait(e, rows_ref, sem):
        for cp in copies(e, rows_ref, sem):
            cp.wait()

    def accum_into(rows_ref, e):
        def body(j, acc):
            return tuple(
                acc[k] + plsc.bitcast(rows_ref[j, pl.ds(16 * k, 16)],
                                      jnp.bfloat16)
                for k in range(2))

        acc = lax.fori_loop(
            0, SEQ, body,
            tuple(jnp.zeros((32,), jnp.bfloat16) for _ in range(2)),
            unroll=8)
        inv = jnp.bfloat16(1.0 / SEQ)
        for k in range(2):
            pooled_v[e, pl.ds(32 * k, 32)] = acc[k] * inv

    bufs = (rows_a, rows_b, rows_c, rows_d)
    sems = (sem_a, sem_b, sem_c, sem_d)

    def group(gi, carry):
        g0 = base + gi * _G
        pltpu.sync_copy(x_hbm.at[pl.ds(g0, _G), :], idx_v)
        for n in range(3):
            start(n, bufs[n], sems[n])

        def quad(p, c):
            e = 4 * p
            start(e + 3, bufs[3], sems[3])
            for n in range(4):
                wait(e + n, bufs[n], sems[n])
                accum_into(bufs[n], e + n)
                if n < 3:
                    @pl.when(p + 1 < _G // 4)
                    def _():
                        start(e + 4 + n, bufs[n], sems[n])
            return c

        lax.fori_loop(0, _G // 4, quad, 0)
        pltpu.sync_copy(pooled_v, out_hbm.at[pl.ds(g0, _G), :])
        return carry

    lax.fori_loop(0, _NG, group, 0)


_pool = functools.partial(
    pl.kernel,
    mesh=plsc.VectorSubcoreMesh(core_axis_name="c", subcore_axis_name="s"),
    out_type=jax.ShapeDtypeStruct((BATCH, EMB), jnp.bfloat16),
    scratch_types=[
        pltpu.VMEM((_G, SEQ), jnp.int32),
        pltpu.VMEM((SEQ, 32), jnp.float32),
        pltpu.VMEM((SEQ, 32), jnp.float32),
        pltpu.VMEM((SEQ, 32), jnp.float32),
        pltpu.VMEM((SEQ, 32), jnp.float32),
        pltpu.VMEM((_G, EMB), jnp.bfloat16),
        pltpu.SemaphoreType.DMA,
        pltpu.SemaphoreType.DMA,
        pltpu.SemaphoreType.DMA,
        pltpu.SemaphoreType.DMA,
    ],
    compiler_params=pltpu.CompilerParams(use_tc_tiling_on_sc=False,
                                         needs_layout_passes=False),
)(_pool_body)


_BBLK = 2048


def _mlp_body(p_ref, w1_ref, b1_ref, w2_ref, b2_ref, o_ref):
    p = p_ref[...].astype(jnp.float32)
    h = jnp.dot(p, w1_ref[...], preferred_element_type=jnp.float32)
    h = jnp.maximum(h + b1_ref[...], 0.0)
    z = jnp.dot(h, w2_ref[...], preferred_element_type=jnp.float32) + b2_ref[...]
    o_ref[...] = 1.0 / (1.0 + jnp.exp(-z))


def _mlp(pooled, W1p, b1, W2, b2):
    return pl.pallas_call(
        _mlp_body,
        grid=(BATCH // _BBLK,),
        in_specs=[
            pl.BlockSpec((_BBLK, EMB), lambda i: (i, 0)),
            pl.BlockSpec((EMB, HID), lambda i: (0, 0)),
            pl.BlockSpec((1, HID), lambda i: (0, 0)),
            pl.BlockSpec((HID, 1), lambda i: (0, 0)),
            pl.BlockSpec((1, 1), lambda i: (0, 0)),
        ],
        out_specs=pl.BlockSpec((_BBLK, 1), lambda i: (i, 0)),
        out_shape=jax.ShapeDtypeStruct((BATCH, 1), jnp.float32),
    )(pooled, W1p, b1.reshape(1, HID), W2, b2.reshape(1, 1))


def kernel(x, emb, W1, b1, W2, b2):
    embp = _prep(emb.T).reshape(_VP, 32)
    x32 = x.astype(jnp.int32)
    q = x32 // _Q
    xt = 4 * (x32 - q * _Q) + q
    pooled = _pool(xt, embp)
    return _mlp(pooled, W1[jnp.asarray(_PERM)], b1, W2, b2)


# speedup vs baseline: 2.8468x; 1.2806x over previous
import functools

import jax
import jax.numpy as jnp
import numpy as np
from jax import lax
from jax.experimental import pallas as pl
from jax.experimental.pallas import tpu as pltpu
from jax.experimental.pallas import tpu_sc as plsc

VOCAB = 1000000
EMB = 64
HID = 128
BATCH = 16384
SEQ = 200

_NC, _NS = 2, 16
_NW = _NC * _NS
_BPW = BATCH // _NW
_G = 64
_NG = _BPW // _G
_S0 = 128
_S1 = SEQ - _S0

_PERM = np.zeros(EMB, dtype=np.int32)
for _c in range(EMB):
    _half, _cc = divmod(_c, 32)
    _w = 16 * _half + _cc // 2
    _PERM[_c] = _w + 32 * (_cc % 2)

_Q = 250880
_VP = 4 * _Q
_PB = 2560
_PGRID = _Q // _PB


def _prep_body(e0_ref, e1_ref, e2_ref, e3_ref, o_ref):
    ws = []
    for ref in (e0_ref, e1_ref, e2_ref, e3_ref):
        u = lax.bitcast_convert_type(ref[...], jnp.uint32)
        ws.append((u[32:, :] & jnp.uint32(0xFFFF0000)) | (u[:32, :] >> 16))
    w = jnp.concatenate(ws, axis=0)
    o_ref[...] = jnp.transpose(
        lax.bitcast_convert_type(w, jnp.float32))


def _prep(et):
    last = VOCAB // _PB
    specs = [
        pl.BlockSpec((EMB, _PB), functools.partial(
            lambda q, i: (0, jnp.minimum(i + _PGRID * q, last)), q))
        for q in range(4)
    ]
    return pl.pallas_call(
        _prep_body,
        grid=(_PGRID,),
        in_specs=specs,
        out_specs=pl.BlockSpec((_PB, 128), lambda i: (i, 0)),
        out_shape=jax.ShapeDtypeStruct((_Q, 128), jnp.float32),
    )(et, et, et, et)


def _pool_body(x_hbm, emb_hbm, out_hbm, idx_v, rows_a, rows_b, rows_c,
               rows_d, pooled_v, sem_a, sem_b, sem_c, sem_d):
    wid = lax.axis_index("s") * _NC + lax.axis_index("c")
    base = wid * _BPW

    def copies(e, rows_ref, sem):
        return (
            pltpu.make_async_copy(
                emb_hbm.at[idx_v.at[e, pl.ds(0, _S0)]],
                rows_ref.at[pl.ds(0, _S0), :], sem),
            pltpu.make_async_copy(
                emb_hbm.at[idx_v.at[e, pl.ds(_S0, _S1)]],
                rows_ref.at[pl.ds(_S0, _S1), :], sem),
        )

    def start(e, rows_ref, sem):
        for cp in copies(e, rows_ref, sem):
            cp.start()

    def wait(e, rows_ref, sem):
        for cp in copies(e, rows_ref, sem):
            cp.wait()

    def accum_into(rows_ref, e):
        def body(j, acc):
            return tuple(
                acc[k] + plsc.bitcast(rows_ref[j, pl.ds(16 * k, 16)],
                                      jnp.bfloat16)
                for k in range(2))

        acc = lax.fori_loop(
            0, SEQ, body,
            tuple(jnp.zeros((32,), jnp.bfloat16) for _ in range(2)),
            unroll=8)
        inv = jnp.bfloat16(1.0 / SEQ)
        for k in range(2):
            pooled_v[e, pl.ds(32 * k, 32)] = acc[k] * inv

    bufs = (rows_a, rows_b, rows_c, rows_d)
    sems = (sem_a, sem_b, sem_c, sem_d)

    def group(gi, carry):
        g0 = base + gi * _G
        pltpu.sync_copy(x_hbm.at[pl.ds(g0, _G), :], idx_v)
        for n in range(3):
            start(n, bufs[n], sems[n])

        def quad(p, c):
            e = 4 * p
            start(e + 3, bufs[3], sems[3])
            for n in range(4):
                wait(e + n, bufs[n], sems[n])
                accum_into(bufs[n], e + n)
                if n < 3:
                    @pl.when(p + 1 < _G // 4)
                    def _():
                        start(e + 4 + n, bufs[n], sems[n])
            return c

        lax.fori_loop(0, _G // 4, quad, 0)
        pltpu.sync_copy(pooled_v, out_hbm.at[pl.ds(g0, _G), :])
        return carry

    lax.fori_loop(0, _NG, group, 0)


_pool = functools.partial(
    pl.kernel,
    mesh=plsc.VectorSubcoreMesh(core_axis_name="c", subcore_axis_name="s"),
    out_type=jax.ShapeDtypeStruct((BATCH, EMB), jnp.bfloat16),
    scratch_types=[
        pltpu.VMEM((_G, SEQ), jnp.int32),
        pltpu.VMEM((SEQ, 32), jnp.float32),
        pltpu.VMEM((SEQ, 32), jnp.float32),
        pltpu.VMEM((SEQ, 32), jnp.float32),
        pltpu.VMEM((SEQ, 32), jnp.float32),
        pltpu.VMEM((_G, EMB), jnp.bfloat16),
        pltpu.SemaphoreType.DMA,
        pltpu.SemaphoreType.DMA,
        pltpu.SemaphoreType.DMA,
        pltpu.SemaphoreType.DMA,
    ],
    compiler_params=pltpu.CompilerParams(use_tc_tiling_on_sc=False,
                                         needs_layout_passes=False),
)(_pool_body)


_BBLK = 2048


def _mlp_body(p_ref, w1_ref, b1_ref, w2_ref, b2_ref, o_ref):
    p = p_ref[...].astype(jnp.float32)
    h = jnp.dot(p, w1_ref[...], preferred_element_type=jnp.float32)
    h = jnp.maximum(h + b1_ref[...], 0.0)
    z = jnp.dot(h, w2_ref[...], preferred_element_type=jnp.float32) + b2_ref[...]
    o_ref[...] = 1.0 / (1.0 + jnp.exp(-z))


def _mlp(pooled, W1p, b1, W2, b2):
    return pl.pallas_call(
        _mlp_body,
        grid=(BATCH // _BBLK,),
        in_specs=[
            pl.BlockSpec((_BBLK, EMB), lambda i: (i, 0)),
            pl.BlockSpec((EMB, HID), lambda i: (0, 0)),
            pl.BlockSpec((1, HID), lambda i: (0, 0)),
            pl.BlockSpec((HID, 1), lambda i: (0, 0)),
            pl.BlockSpec((1, 1), lambda i: (0, 0)),
        ],
        out_specs=pl.BlockSpec((_BBLK, 1), lambda i: (i, 0)),
        out_shape=jax.ShapeDtypeStruct((BATCH, 1), jnp.float32),
    )(pooled, W1p, b1.reshape(1, HID), W2, b2.reshape(1, 1))


def kernel(x, emb, W1, b1, W2, b2):
    embp = _prep(emb.T).reshape(_VP, 32)
    x32 = x.astype(jnp.int32)
    q = x32 // _Q
    xt = 4 * (x32 - q * _Q) + q
    pooled = _pool(xt, embp)
    return _mlp(pooled, W1[jnp.asarray(_PERM)], b1, W2, b2)


# speedup vs baseline: 2.9966x; 1.0526x over previous
import functools

import jax
import jax.numpy as jnp
import numpy as np
from jax import lax
from jax.experimental import pallas as pl
from jax.experimental.pallas import tpu as pltpu
from jax.experimental.pallas import tpu_sc as plsc

VOCAB = 1000000
EMB = 64
HID = 128
BATCH = 16384
SEQ = 200

_NC, _NS = 2, 16
_NW = _NC * _NS
_BPW = BATCH // _NW
_G = 64
_NG = _BPW // _G
_S0 = 128
_S1 = SEQ - _S0

_PERM = np.zeros(EMB, dtype=np.int32)
for _c in range(EMB):
    _half, _cc = divmod(_c, 32)
    _w = 16 * _half + _cc // 2
    _PERM[_c] = _w + 32 * (_cc % 2)

_Q = 250880
_VP = 4 * _Q
_PB = 2560
_PGRID = _Q // _PB


def _prep_body(e0_ref, e1_ref, e2_ref, e3_ref, o_ref):
    ws = []
    for ref in (e0_ref, e1_ref, e2_ref, e3_ref):
        u = lax.bitcast_convert_type(ref[...], jnp.uint32)
        ws.append((u[32:, :] & jnp.uint32(0xFFFF0000)) | (u[:32, :] >> 16))
    w = jnp.concatenate(ws, axis=0)
    o_ref[...] = jnp.transpose(
        lax.bitcast_convert_type(w, jnp.float32))


def _prep(et):
    last = VOCAB // _PB
    specs = [
        pl.BlockSpec((EMB, _PB), functools.partial(
            lambda q, i: (0, jnp.minimum(i + _PGRID * q, last)), q))
        for q in range(4)
    ]
    return pl.pallas_call(
        _prep_body,
        grid=(_PGRID,),
        in_specs=specs,
        out_specs=pl.BlockSpec((_PB, 128), lambda i: (i, 0)),
        out_shape=jax.ShapeDtypeStruct((_Q, 128), jnp.float32),
    )(et, et, et, et)


_NBUF = 8


def _pool_body(x_hbm, emb_hbm, out_hbm, idx_v, rows, pooled_v, sems):
    wid = lax.axis_index("s") * _NC + lax.axis_index("c")
    base = wid * _BPW

    def copies(e, rows_ref, sem):
        return (
            pltpu.make_async_copy(
                emb_hbm.at[idx_v.at[e, pl.ds(0, _S0)]],
                rows_ref.at[pl.ds(0, _S0), :], sem),
            pltpu.make_async_copy(
                emb_hbm.at[idx_v.at[e, pl.ds(_S0, _S1)]],
                rows_ref.at[pl.ds(_S0, _S1), :], sem),
        )

    def start(e, rows_ref, sem):
        for cp in copies(e, rows_ref, sem):
            cp.start()

    def wait(e, rows_ref, sem):
        for cp in copies(e, rows_ref, sem):
            cp.wait()

    def accum_into(rows_ref, e):
        def body(j, acc):
            return tuple(
                acc[k] + plsc.bitcast(rows_ref[j, pl.ds(16 * k, 16)],
                                      jnp.bfloat16)
                for k in range(2))

        acc = lax.fori_loop(
            0, SEQ, body,
            tuple(jnp.zeros((32,), jnp.bfloat16) for _ in range(2)),
            unroll=8)
        inv = jnp.bfloat16(1.0 / SEQ)
        for k in range(2):
            pooled_v[e, pl.ds(32 * k, 32)] = acc[k] * inv

    def group(gi, carry):
        g0 = base + gi * _G
        pltpu.sync_copy(x_hbm.at[pl.ds(g0, _G), :], idx_v)
        for n in range(_NBUF - 1):
            start(n, rows[n], sems[n])

        def ring(p, c):
            e = _NBUF * p
            start(e + _NBUF - 1, rows[_NBUF - 1], sems[_NBUF - 1])
            for n in range(_NBUF):
                wait(e + n, rows[n], sems[n])
                accum_into(rows[n], e + n)
                if n < _NBUF - 1:
                    @pl.when(p + 1 < _G // _NBUF)
                    def _():
                        start(e + _NBUF + n, rows[n], sems[n])
            return c

        lax.fori_loop(0, _G // _NBUF, ring, 0)
        pltpu.sync_copy(pooled_v, out_hbm.at[pl.ds(g0, _G), :])
        return carry

    lax.fori_loop(0, _NG, group, 0)


_pool = functools.partial(
    pl.kernel,
    mesh=plsc.VectorSubcoreMesh(core_axis_name="c", subcore_axis_name="s"),
    out_type=jax.ShapeDtypeStruct((BATCH, EMB), jnp.bfloat16),
    scratch_types=[
        pltpu.VMEM((_G, SEQ), jnp.int32),
        [pltpu.VMEM((SEQ, 32), jnp.float32)] * _NBUF,
        pltpu.VMEM((_G, EMB), jnp.bfloat16),
        [pltpu.SemaphoreType.DMA] * _NBUF,
    ],
    compiler_params=pltpu.CompilerParams(use_tc_tiling_on_sc=False,
                                         needs_layout_passes=False),
)(_pool_body)


_BBLK = 2048


def _mlp_body(p_ref, w1_ref, b1_ref, w2_ref, b2_ref, o_ref):
    p = p_ref[...].astype(jnp.float32)
    h = jnp.dot(p, w1_ref[...], preferred_element_type=jnp.float32)
    h = jnp.maximum(h + b1_ref[...], 0.0)
    z = jnp.dot(h, w2_ref[...], preferred_element_type=jnp.float32) + b2_ref[...]
    o_ref[...] = 1.0 / (1.0 + jnp.exp(-z))


def _mlp(pooled, W1p, b1, W2, b2):
    return pl.pallas_call(
        _mlp_body,
        grid=(BATCH // _BBLK,),
        in_specs=[
            pl.BlockSpec((_BBLK, EMB), lambda i: (i, 0)),
            pl.BlockSpec((EMB, HID), lambda i: (0, 0)),
            pl.BlockSpec((1, HID), lambda i: (0, 0)),
            pl.BlockSpec((HID, 1), lambda i: (0, 0)),
            pl.BlockSpec((1, 1), lambda i: (0, 0)),
        ],
        out_specs=pl.BlockSpec((_BBLK, 1), lambda i: (i, 0)),
        out_shape=jax.ShapeDtypeStruct((BATCH, 1), jnp.float32),
    )(pooled, W1p, b1.reshape(1, HID), W2, b2.reshape(1, 1))


def kernel(x, emb, W1, b1, W2, b2):
    x32 = x.astype(jnp.int32)
    q = x32 // _Q
    xt = 4 * (x32 - q * _Q) + q
    et, xt = lax.optimization_barrier((emb.T, xt))
    embp = _prep(et).reshape(_VP, 32)
    pooled = _pool(xt, embp)
    return _mlp(pooled, W1[jnp.asarray(_PERM)], b1, W2, b2)
